# baseline (speedup 1.0000x reference)
import math

import jax
import jax.numpy as jnp
from jax.experimental import pallas as pl
from jax.experimental.pallas import tpu as pltpu

DIM = 256
HEADS = 8
HEAD_DIM = DIM // HEADS
SCALE = HEAD_DIM ** (-0.5)
THRESH = 15.0
N_TOK = 2048
LOG2E = math.log2(math.e)

_NAN = float("nan")


def _body(x_ref, wqkv_ref, wp_ref, b_ref, o_ref,
          qkvT_ref, v01_ref, nan_ref, outT_ref):
    i = pl.program_id(0)

    @pl.when(i == 0)
    def _prep():
        x = x_ref[...]
        w = wqkv_ref[...]
        qkvT = jax.lax.dot_general(
            w.astype(jnp.bfloat16), x.astype(jnp.bfloat16),
            (((1,), (1,)), ((), ())),
            preferred_element_type=jnp.float32)
        nrm = jnp.sqrt(jnp.sum(x * x, axis=-1, keepdims=True))
        mx = jnp.max(nrm)
        valid = (nrm / mx * 75.0) >= THRESH
        nan_ref[...] = jnp.where(valid, 0.0, _NAN).astype(jnp.float32)
        v01row = jnp.where(valid, 1.0, 0.0).reshape(1, N_TOK)
        v01_ref[...] = v01row.astype(jnp.bfloat16)
        q = qkvT[:DIM, :] * (SCALE * LOG2E)
        k = qkvT[DIM:2 * DIM, :]
        v = qkvT[2 * DIM:, :] * v01row
        qkv = jnp.concatenate([q, k, v], axis=0).astype(jnp.bfloat16)
        qkvT_ref[...] = qkv.reshape(3 * HEADS, HEAD_DIM, N_TOK)

    @pl.when(jnp.logical_and(i >= 1, i <= HEADS))
    def _attn():
        h = i - 1
        q = qkvT_ref[h]
        k = qkvT_ref[HEADS + h]
        v = qkvT_ref[2 * HEADS + h]
        qf = q.astype(jnp.float32)
        kf = k.astype(jnp.float32)
        qn = jnp.sqrt(jnp.sum(qf * qf, axis=0, keepdims=True))
        kn2 = jnp.sum(kf * kf, axis=0, keepdims=True)
        kmax = jnp.sqrt(jnp.max(kn2))
        m = qn * kmax
        zeros7 = jnp.zeros((7, N_TOK), jnp.bfloat16)
        q_aug = jnp.concatenate(
            [q, (-m).astype(jnp.bfloat16), zeros7], axis=0)
        k_aug = jnp.concatenate(
            [k, jnp.ones((1, N_TOK), jnp.bfloat16), zeros7], axis=0)
        vcat = jnp.concatenate(
            [v, jnp.broadcast_to(v01_ref[...], (8, N_TOK))], axis=0)
        CB = N_TOK // 8
        oa = jnp.zeros((HEAD_DIM + 8, N_TOK), jnp.float32)
        for cb in range(8):
            ksl = k_aug[:, cb * CB:(cb + 1) * CB]
            t = jax.lax.dot_general(
                q_aug, ksl, (((0,), (0,)), ((), ())),
                preferred_element_type=jnp.float32)
            e = jnp.exp2(t.astype(jnp.bfloat16))
            vsl = vcat[:, cb * CB:(cb + 1) * CB]
            oa = oa + jax.lax.dot_general(
                vsl, e, (((1,), (1,)), ((), ())),
                preferred_element_type=jnp.float32)
        o = oa[:HEAD_DIM, :] / oa[HEAD_DIM:HEAD_DIM + 1, :]
        outT_ref[h] = o.astype(jnp.bfloat16)

    @pl.when(i == HEADS + 1)
    def _fin():
        outT = outT_ref[...].reshape(DIM, N_TOK)
        res = jax.lax.dot_general(
            outT, wp_ref[...].astype(jnp.bfloat16),
            (((0,), (1,)), ((), ())),
            preferred_element_type=jnp.float32)
        o_ref[...] = res + b_ref[...] + x_ref[...] + nan_ref[...]


@jax.jit
def kernel(x, W_qkv, W_proj, b_proj):
    B, T, H, W, C = x.shape
    N = T * H * W
    x_seq = x.reshape(N, C)

    out = pl.pallas_call(
        _body,
        grid=(HEADS + 2,),
        in_specs=[
            pl.BlockSpec((N, C), lambda i: (0, 0)),
            pl.BlockSpec((3 * C, C), lambda i: (0, 0)),
            pl.BlockSpec((C, C), lambda i: (0, 0)),
            pl.BlockSpec((1, C), lambda i: (0, 0)),
        ],
        out_specs=pl.BlockSpec((N, C), lambda i: (0, 0)),
        out_shape=jax.ShapeDtypeStruct((N, C), jnp.float32),
        scratch_shapes=[
            pltpu.VMEM((3 * HEADS, HEAD_DIM, N_TOK), jnp.bfloat16),
            pltpu.VMEM((1, N_TOK), jnp.bfloat16),
            pltpu.VMEM((N_TOK, 1), jnp.float32),
            pltpu.VMEM((HEADS, HEAD_DIM, N_TOK), jnp.bfloat16),
        ],
    )(x_seq, W_qkv, W_proj, b_proj.reshape(1, C))

    return out.reshape(B, T, H, W, C)


# speedup vs baseline: 1.1446x; 1.1446x over previous
import math

import jax
import jax.numpy as jnp
from jax.experimental import pallas as pl
from jax.experimental.pallas import tpu as pltpu

DIM = 256
HEADS = 8
HEAD_DIM = DIM // HEADS
SCALE = HEAD_DIM ** (-0.5)
THRESH = 15.0
N_TOK = 2048
LOG2E = math.log2(math.e)

_NAN = float("nan")


def _body(x_ref, wqkv_ref, wp_ref, b_ref, o_ref,
          qkvT_ref, v01_ref, nan_ref, outT_ref):
    i = pl.program_id(0)

    @pl.when(i == 0)
    def _prep():
        x = x_ref[...]
        w = wqkv_ref[...]
        qkvT = jax.lax.dot_general(
            w.astype(jnp.bfloat16), x.astype(jnp.bfloat16),
            (((1,), (1,)), ((), ())),
            preferred_element_type=jnp.float32)
        nrm = jnp.sqrt(jnp.sum(x * x, axis=-1, keepdims=True))
        mx = jnp.max(nrm)
        valid = (nrm / mx * 75.0) >= THRESH
        nan_ref[...] = jnp.where(valid, 0.0, _NAN).astype(jnp.float32)
        v01row = jnp.where(valid, 1.0, 0.0).reshape(1, N_TOK)
        v01_ref[...] = v01row.astype(jnp.bfloat16)
        q = qkvT[:DIM, :] * (SCALE * LOG2E)
        k = qkvT[DIM:2 * DIM, :]
        v = qkvT[2 * DIM:, :] * v01row
        qkv = jnp.concatenate([q, k, v], axis=0).astype(jnp.bfloat16)
        qkvT_ref[...] = qkv.reshape(3 * HEADS, HEAD_DIM, N_TOK)

    @pl.when(jnp.logical_and(i >= 1, i <= HEADS // 2))
    def _attn():
      for dh in range(2):
          h = (i - 1) * 2 + dh
          q = qkvT_ref[h]
          k = qkvT_ref[HEADS + h]
          v = qkvT_ref[2 * HEADS + h]
          qf = q.astype(jnp.float32)
          kf = k.astype(jnp.float32)
          qn = jnp.sqrt(jnp.sum(qf * qf, axis=0, keepdims=True))
          kn2 = jnp.sum(kf * kf, axis=0, keepdims=True)
          kmax = jnp.sqrt(jnp.max(kn2))
          m = qn * kmax
          zeros7 = jnp.zeros((7, N_TOK), jnp.bfloat16)
          q_aug = jnp.concatenate(
              [q, (-m).astype(jnp.bfloat16), zeros7], axis=0)
          k_aug = jnp.concatenate(
              [k, jnp.ones((1, N_TOK), jnp.bfloat16), zeros7], axis=0)
          vcat = jnp.concatenate(
              [v, jnp.broadcast_to(v01_ref[...], (8, N_TOK))], axis=0)
          CB = N_TOK // 4
          oa = jnp.zeros((HEAD_DIM + 8, N_TOK), jnp.float32)
          for cb in range(4):
              ksl = k_aug[:, cb * CB:(cb + 1) * CB]
              t = jax.lax.dot_general(
                  q_aug, ksl, (((0,), (0,)), ((), ())),
                  preferred_element_type=jnp.float32)
              e = jnp.exp2(t.astype(jnp.bfloat16))
              vsl = vcat[:, cb * CB:(cb + 1) * CB]
              oa = oa + jax.lax.dot_general(
                  vsl, e, (((1,), (1,)), ((), ())),
                  preferred_element_type=jnp.float32)
          o = oa[:HEAD_DIM, :] / oa[HEAD_DIM:HEAD_DIM + 1, :]
          outT_ref[h] = o.astype(jnp.bfloat16)

    @pl.when(i == HEADS // 2 + 1)
    def _fin():
        outT = outT_ref[...].reshape(DIM, N_TOK)
        res = jax.lax.dot_general(
            outT, wp_ref[...].astype(jnp.bfloat16),
            (((0,), (1,)), ((), ())),
            preferred_element_type=jnp.float32)
        o_ref[...] = res + b_ref[...] + x_ref[...] + nan_ref[...]


@jax.jit
def kernel(x, W_qkv, W_proj, b_proj):
    B, T, H, W, C = x.shape
    N = T * H * W
    x_seq = x.reshape(N, C)

    out = pl.pallas_call(
        _body,
        grid=(HEADS // 2 + 2,),
        in_specs=[
            pl.BlockSpec((N, C), lambda i: (0, 0)),
            pl.BlockSpec((3 * C, C), lambda i: (0, 0)),
            pl.BlockSpec((C, C), lambda i: (0, 0)),
            pl.BlockSpec((1, C), lambda i: (0, 0)),
        ],
        out_specs=pl.BlockSpec((N, C), lambda i: (0, 0)),
        out_shape=jax.ShapeDtypeStruct((N, C), jnp.float32),
        scratch_shapes=[
            pltpu.VMEM((3 * HEADS, HEAD_DIM, N_TOK), jnp.bfloat16),
            pltpu.VMEM((1, N_TOK), jnp.bfloat16),
            pltpu.VMEM((N_TOK, 1), jnp.float32),
            pltpu.VMEM((HEADS, HEAD_DIM, N_TOK), jnp.bfloat16),
        ],
    )(x_seq, W_qkv, W_proj, b_proj.reshape(1, C))

    return out.reshape(B, T, H, W, C)
